# Initial kernel scaffold; baseline (speedup 1.0000x reference)
import functools

import jax
import jax.numpy as jnp
from jax import lax
from jax.experimental import pallas as pl
from jax.experimental.pallas import tpu as pltpu
from jax.experimental.pallas import tpu_sc as plsc

D = 64
NC = 2
NS = 16
NW = NC * NS

GROUP = 128
CHUNK_G = 4
CHUNK = GROUP * CHUNK_G


@functools.partial(jax.jit, static_argnames=("n_total",))
def _sc_gather(idx_flat, table, n_total):
    npw = n_total // NW
    chunks_per_w = npw // CHUNK
    mesh = plsc.VectorSubcoreMesh(core_axis_name="c", subcore_axis_name="s")

    @functools.partial(
        pl.kernel,
        mesh=mesh,
        out_type=jax.ShapeDtypeStruct((n_total, D), jnp.float32),
        scratch_types=[
            pltpu.VMEM((CHUNK_G, GROUP), jnp.int32),
            pltpu.VMEM((CHUNK, D), jnp.float32),
            pltpu.SemaphoreType.DMA,
        ],
    )
    def k(idx_hbm, table_hbm, out_hbm, idx_v, rows_v, gsem):
        wid = lax.axis_index("s") * NC + lax.axis_index("c")
        wbase = wid * npw

        @pl.loop(0, chunks_per_w)
        def _chunk(g):
            base = wbase + g * CHUNK
            pltpu.sync_copy(
                idx_hbm.at[pl.ds(base, CHUNK)],
                idx_v.at[...],
            )
            for j in range(CHUNK_G):
                for i in range(GROUP // 16):
                    sl = pl.ds(i * 16, 16)
                    idx_v[j, sl] = idx_v[j, sl] + 1
            for j in range(CHUNK_G):
                pltpu.async_copy(
                    table_hbm.at[idx_v.at[j]],
                    rows_v.at[pl.ds(j * GROUP, GROUP), :],
                    gsem,
                )
            for j in range(CHUNK_G):
                pltpu.make_async_copy(
                    table_hbm.at[idx_v.at[j]],
                    rows_v.at[pl.ds(j * GROUP, GROUP), :],
                    gsem,
                ).wait()
            pltpu.sync_copy(
                rows_v.at[...],
                out_hbm.at[pl.ds(base, CHUNK), :],
            )

    return k(idx_flat, table)


def kernel(channel_idx, table):
    B, L = channel_idx.shape
    n_total = B * L
    idx_flat = channel_idx.reshape(n_total).astype(jnp.int32)
    out = _sc_gather(idx_flat, table, n_total)
    return out.reshape(B, L, 1, D)


# speedup vs baseline: 4.9891x; 4.9891x over previous
import functools

import jax
import jax.numpy as jnp
from jax import lax
from jax.experimental import pallas as pl
from jax.experimental.pallas import tpu as pltpu
from jax.experimental.pallas import tpu_sc as plsc

D = 64
NC = 2
NS = 16
NW = NC * NS

GROUP = 128
CHUNK_G = 8
CHUNK = GROUP * CHUNK_G


@functools.partial(jax.jit, static_argnames=("n_total",))
def _sc_gather(idx_flat, table, n_total):
    npw = n_total // NW
    chunks_per_w = npw // CHUNK
    mesh = plsc.VectorSubcoreMesh(core_axis_name="c", subcore_axis_name="s")

    @functools.partial(
        pl.kernel,
        mesh=mesh,
        compiler_params=pltpu.CompilerParams(use_tc_tiling_on_sc=False),
        out_type=jax.ShapeDtypeStruct((n_total, D), jnp.float32),
        scratch_types=[
            pltpu.VMEM((CHUNK_G, GROUP), jnp.int32),
            pltpu.VMEM((CHUNK, D), jnp.float32),
            pltpu.SemaphoreType.DMA,
        ],
    )
    def k(idx_hbm, table_hbm, out_hbm, idx_v, rows_v, gsem):
        wid = lax.axis_index("s") * NC + lax.axis_index("c")
        wbase = wid * npw

        @pl.loop(0, chunks_per_w)
        def _chunk(g):
            base = wbase + g * CHUNK
            grow = pl.multiple_of(base // GROUP, 8)
            pltpu.sync_copy(
                idx_hbm.at[pl.ds(grow, CHUNK_G), :],
                idx_v.at[...],
            )
            for j in range(CHUNK_G):
                for i in range(GROUP // 16):
                    sl = pl.ds(i * 16, 16)
                    idx_v[j, sl] = idx_v[j, sl] + 1
            copies = []
            for j in range(CHUNK_G):
                copies.append(pltpu.async_copy(
                    table_hbm.at[idx_v.at[j]],
                    rows_v.at[pl.ds(j * GROUP, GROUP), :],
                    gsem,
                ))
            for c in copies:
                c.wait()
            pltpu.sync_copy(
                rows_v.at[...],
                out_hbm.at[pl.ds(base, CHUNK), :],
            )

    return k(idx_flat, table)


def kernel(channel_idx, table):
    B, L = channel_idx.shape
    n_total = B * L
    idx_flat = channel_idx.reshape(n_total // GROUP, GROUP).astype(jnp.int32)
    out = _sc_gather(idx_flat, table, n_total)
    return out.reshape(B, L, 1, D)


# speedup vs baseline: 5.1831x; 1.0389x over previous
import functools

import jax
import jax.numpy as jnp
from jax import lax
from jax.experimental import pallas as pl
from jax.experimental.pallas import tpu as pltpu
from jax.experimental.pallas import tpu_sc as plsc

D = 64
NC = 2
NS = 16
NW = NC * NS

GROUP = 128
CHUNK_G = 4
CHUNK = GROUP * CHUNK_G


@functools.partial(jax.jit, static_argnames=("n_total",))
def _sc_gather(idx_flat, table, n_total):
    npw = n_total // NW
    chunks_per_w = npw // CHUNK
    assert chunks_per_w % 2 == 0
    mesh = plsc.VectorSubcoreMesh(core_axis_name="c", subcore_axis_name="s")

    @functools.partial(
        pl.kernel,
        mesh=mesh,
        compiler_params=pltpu.CompilerParams(use_tc_tiling_on_sc=False),
        out_type=jax.ShapeDtypeStruct((n_total, D), jnp.float32),
        scratch_types=[
            pltpu.VMEM((CHUNK,), jnp.int32),
            pltpu.VMEM((CHUNK,), jnp.int32),
            pltpu.VMEM((CHUNK, D), jnp.float32),
            pltpu.VMEM((CHUNK, D), jnp.float32),
            pltpu.SemaphoreType.DMA,
            pltpu.SemaphoreType.DMA,
            pltpu.SemaphoreType.DMA,
            pltpu.SemaphoreType.DMA,
        ],
    )
    def k(idx_hbm, table_hbm, out_hbm, i0, i1, r0, r1, g0, g1, w0, w1):
        ibuf = (i0, i1)
        rbuf = (r0, r1)
        gsem = (g0, g1)
        wsem = (w0, w1)
        wid = lax.axis_index("s") * NC + lax.axis_index("c")
        wbase = wid * npw

        def load_fix(c, b):
            pltpu.sync_copy(
                idx_hbm.at[pl.ds(wbase + c * CHUNK, CHUNK)], ibuf[b].at[...]
            )
            for i in range(CHUNK // 16):
                sl = pl.ds(i * 16, 16)
                ibuf[b][sl] = ibuf[b][sl] + 1

        def fire_gathers(b):
            for j in range(CHUNK_G):
                pltpu.async_copy(
                    table_hbm.at[ibuf[b].at[pl.ds(j * GROUP, GROUP)]],
                    rbuf[b].at[pl.ds(j * GROUP, GROUP), :],
                    gsem[b],
                )

        def drain_gathers(b):
            for j in range(CHUNK_G):
                pltpu.make_async_copy(
                    table_hbm.at[ibuf[b].at[pl.ds(j * GROUP, GROUP)]],
                    rbuf[b].at[pl.ds(j * GROUP, GROUP), :],
                    gsem[b],
                ).wait()

        def fire_out(c, b):
            pltpu.async_copy(
                rbuf[b].at[...],
                out_hbm.at[pl.ds(wbase + c * CHUNK, CHUNK), :],
                wsem[b],
            )

        def wait_out(c, b):
            pltpu.make_async_copy(
                rbuf[b].at[...],
                out_hbm.at[pl.ds(wbase + c * CHUNK, CHUNK), :],
                wsem[b],
            ).wait()

        load_fix(0, 0)
        fire_gathers(0)

        @pl.loop(0, chunks_per_w, step=2)
        def _pair(g):
            for b in range(2):
                c = g + b
                nxt = c + 1

                @pl.when(nxt < chunks_per_w)
                def _prefetch():
                    load_fix(nxt, 1 - b)

                @pl.when(c >= 1)
                def _free():
                    wait_out(c - 1, 1 - b)

                @pl.when(nxt < chunks_per_w)
                def _fire():
                    fire_gathers(1 - b)

                drain_gathers(b)
                fire_out(c, b)

        wait_out(chunks_per_w - 1, 1)

    return k(idx_flat, table)


def kernel(channel_idx, table):
    B, L = channel_idx.shape
    n_total = B * L
    idx_flat = channel_idx.reshape(n_total).astype(jnp.int32)
    out = _sc_gather(idx_flat, table, n_total)
    return out.reshape(B, L, 1, D)
